# baseline (speedup 1.0000x reference)
import jax
import jax.numpy as jnp
from jax.experimental import pallas as pl

_B, _C, _L = 8, 4096, 1024
_FLAT = _C * _L
_E16 = 1.1920928955078125e-07
_RB = 512
_CHUNK = 512
_NCH = _B * _C // _CHUNK
_CPB = _C // _CHUNK


def _topk_body(x_ref, val_ref, key_ref, col_ref):
    i = pl.program_id(0)
    x = x_ref[...]
    m = jnp.max(x, axis=1, keepdims=True)
    io = jax.lax.broadcasted_iota(jnp.int32, x.shape, 1)
    am = jnp.min(jnp.where(x == m, io, _L), axis=1, keepdims=True)
    b = i // (_C // _RB)
    val_ref[...] = m
    key_ref[...] = am * _L + b * _FLAT
    col_ref[...] = x[:, 0:1]


def _stats_body(sk_ref, nx_ref, sv_ref, yp0_ref, yt0_ref, o_ref):
    sub8 = jax.lax.broadcasted_iota(jnp.int32, (_CHUNK, 8), 1)
    lane128 = jax.lax.broadcasted_iota(jnp.int32, (_CHUNK, 128), 1)
    t2d = (jax.lax.broadcasted_iota(jnp.int32, (8, 128), 0) * 128
           + jax.lax.broadcasted_iota(jnp.int32, (8, 128), 1))

    tp32 = jnp.zeros((8, 128), jnp.float32)
    fp32 = jnp.zeros((8, 128), jnp.float32)
    fn32 = jnp.zeros((8, 128), jnp.float32)
    for b in range(_B):
        acc = jnp.zeros((8, 128), jnp.float32)
        for k in range(_CPB):
            j = b * _CPB + k
            ck = sk_ref[:, j:j + 1]
            nk = nx_ref[:, j:j + 1]
            vk = sv_ref[:, j:j + 1]
            ts = (ck >> 10) & (_L - 1)
            w = jnp.where(nk != ck, vk, 0.0)
            a = jnp.where((ts >> 7) == sub8, w, 0.0)
            bm = ((ts & 127) == lane128).astype(jnp.float32)
            acc = acc + jax.lax.dot_general(
                a, bm, (((0,), (0,)), ((), ())),
                preferred_element_type=jnp.float32)
        s = acc / yp0_ref[8 * b:8 * b + 8, :]
        oh = yt0_ref[b:b + 1, 0:1] == t2d
        tp32 = tp32 + jnp.where(oh, s, 0.0)
        fp32 = fp32 + jnp.where(oh, 0.0, s)
        fn32 = fn32 + jnp.where(oh, 1.0 - s, 0.0)

    def r16(x):
        u = jax.lax.bitcast_convert_type(x, jnp.int32)
        au = u & jnp.int32(0x7FFFFFFF)
        sb = u & jnp.int32(-0x80000000)
        signf = jax.lax.bitcast_convert_type(sb | jnp.int32(0x3F800000),
                                             jnp.float32)
        ex = au >> 23
        m = (au & jnp.int32(0x7FFFFF)) | jnp.int32(0x800000)
        s = jnp.clip(126 - ex, 13, 25)
        low = m & ((1 << s) - 1)
        half = 1 << (s - 1)
        q = m >> s
        q = q + jnp.where((low > half) | ((low == half) & ((q & 1) == 1)), 1, 0)
        p2 = jax.lax.bitcast_convert_type((ex + s - 23) << 23, jnp.float32)
        res = q.astype(jnp.float32) * p2 * signf
        res = jnp.where(au >= jnp.int32(0x477FF000),
                        signf * jnp.float32(jnp.inf), res)
        return jnp.where(ex == 255, x, res)

    e = jnp.float32(_E16)
    tp = r16(tp32)
    fp = r16(fp32)
    fn = r16(fn32)
    prec = r16(tp / r16(r16(tp + fp) + e))
    rec = r16(tp / r16(r16(tp + fn) + e))
    f1 = r16(r16(r16(prec * rec) * jnp.float32(2.0)) / r16(r16(prec + rec) + e))
    f1c = jnp.minimum(jnp.maximum(f1, e), jnp.float32(1.0))

    inv_n = jnp.float32(2.0 ** -22)
    rest = jnp.float32((_C * _L - _L) * 2.0 ** -23)
    mf1 = (jnp.sum(f1c) + rest) * inv_n
    out1 = jnp.sum(prec) * inv_n
    out2 = jnp.sum(rec) * inv_n

    lane = jax.lax.broadcasted_iota(jnp.int32, (8, 128), 1)
    mf1_16 = r16(jnp.where(lane == 0, mf1, 0.0))
    o_ref[...] = jnp.where(lane == 0, jnp.float32(1.0) - mf1_16,
                           jnp.where(lane == 1, out1, out2))


def kernel(y_pred, y_true):
    yp = y_pred.reshape(_B * _C, _L)
    vals, keys, ypcol = pl.pallas_call(
        _topk_body,
        grid=(_B * _C // _RB,),
        in_specs=[pl.BlockSpec((_RB, _L), lambda i: (i, 0))],
        out_specs=[pl.BlockSpec((_RB, 1), lambda i: (i, 0)),
                   pl.BlockSpec((_RB, 1), lambda i: (i, 0)),
                   pl.BlockSpec((_RB, 1), lambda i: (i, 0))],
        out_shape=[jax.ShapeDtypeStruct((_B * _C, 1), jnp.float32),
                   jax.ShapeDtypeStruct((_B * _C, 1), jnp.int32),
                   jax.ShapeDtypeStruct((_B * _C, 1), jnp.float32)],
    )(yp)

    sk, sv = jax.lax.sort_key_val(keys.reshape(_B * _C), vals.reshape(_B * _C),
                                  is_stable=False)
    nx = jnp.concatenate([sk[1:], jnp.full((1,), -1, jnp.int32)])

    skm = sk.reshape(_NCH, _CHUNK).T
    nxm = nx.reshape(_NCH, _CHUNK).T
    svm = sv.reshape(_NCH, _CHUNK).T
    yp0 = ypcol.reshape(_B, _C)[:, :_L].reshape(_B * 8, 128)
    yt0 = y_true[:, :128]

    o = pl.pallas_call(
        _stats_body,
        out_shape=jax.ShapeDtypeStruct((8, 128), jnp.float32),
    )(skm, nxm, svm, yp0, yt0)

    return (o[0, 0].astype(jnp.float16),
            o[0, 1].astype(jnp.float16),
            o[0, 2].astype(jnp.float16))


# speedup vs baseline: 1.1031x; 1.1031x over previous
import jax
import jax.numpy as jnp
from jax.experimental import pallas as pl

_B, _C, _L = 8, 4096, 1024
_FLAT = _C * _L
_E16 = 1.1920928955078125e-07
_RB = 512
_CHUNK = 512
_NCH = _B * _C // _CHUNK
_CPB = _C // _CHUNK


def _topk_body(x_ref, val_ref, key_ref, col_ref):
    i = pl.program_id(0)
    x = x_ref[...]
    m = jnp.max(x, axis=1, keepdims=True)
    io = jax.lax.broadcasted_iota(jnp.int32, x.shape, 1)
    am = jnp.min(jnp.where(x == m, io, _L), axis=1, keepdims=True)
    b = i // (_C // _RB)
    val_ref[...] = m
    key_ref[...] = am * _L + b * _FLAT
    col_ref[...] = x[:, 0:1]


def _stats_body(sk_ref, sv_ref, yp0_ref, yt0_ref, o_ref):
    sub8 = jax.lax.broadcasted_iota(jnp.int32, (8, _CHUNK), 0)
    lane128 = jax.lax.broadcasted_iota(jnp.int32, (128, _CHUNK), 0)
    lane512 = jax.lax.broadcasted_iota(jnp.int32, (1, _CHUNK), 1)
    t2d = (jax.lax.broadcasted_iota(jnp.int32, (8, 128), 0) * 128
           + jax.lax.broadcasted_iota(jnp.int32, (8, 128), 1))

    tp32 = jnp.zeros((8, 128), jnp.float32)
    fp32 = jnp.zeros((8, 128), jnp.float32)
    fn32 = jnp.zeros((8, 128), jnp.float32)
    for b in range(_B):
        acc = jnp.zeros((8, 128), jnp.float32)
        for k in range(_CPB):
            j = b * _CPB + k
            ck = sk_ref[j:j + 1, :]
            vk = sv_ref[j:j + 1, :]
            if j + 1 < _NCH:
                nxt0 = sk_ref[j + 1:j + 2, 0:1]
            else:
                nxt0 = jnp.full((1, 1), -1, jnp.int32)
            nk = jnp.where(lane512 == _CHUNK - 1, nxt0,
                           jnp.roll(ck, -1, axis=1))
            ts = (ck >> 10) & (_L - 1)
            w = jnp.where(nk != ck, vk, 0.0)
            a = jnp.where((ts >> 7) == sub8, w, 0.0)
            bm = ((ts & 127) == lane128).astype(jnp.float32)
            acc = acc + jax.lax.dot_general(
                a, bm, (((1,), (1,)), ((), ())),
                preferred_element_type=jnp.float32)
        s = acc / yp0_ref[8 * b:8 * b + 8, :]
        oh = yt0_ref[b:b + 1, 0:1] == t2d
        tp32 = tp32 + jnp.where(oh, s, 0.0)
        fp32 = fp32 + jnp.where(oh, 0.0, s)
        fn32 = fn32 + jnp.where(oh, 1.0 - s, 0.0)

    def r16(x):
        u = jax.lax.bitcast_convert_type(x, jnp.int32)
        au = u & jnp.int32(0x7FFFFFFF)
        sb = u & jnp.int32(-0x80000000)
        signf = jax.lax.bitcast_convert_type(sb | jnp.int32(0x3F800000),
                                             jnp.float32)
        ex = au >> 23
        m = (au & jnp.int32(0x7FFFFF)) | jnp.int32(0x800000)
        s = jnp.clip(126 - ex, 13, 25)
        low = m & ((1 << s) - 1)
        half = 1 << (s - 1)
        q = m >> s
        q = q + jnp.where((low > half) | ((low == half) & ((q & 1) == 1)), 1, 0)
        p2 = jax.lax.bitcast_convert_type((ex + s - 23) << 23, jnp.float32)
        res = q.astype(jnp.float32) * p2 * signf
        res = jnp.where(au >= jnp.int32(0x477FF000),
                        signf * jnp.float32(jnp.inf), res)
        return jnp.where(ex == 255, x, res)

    e = jnp.float32(_E16)
    tp = r16(tp32)
    fp = r16(fp32)
    fn = r16(fn32)
    prec = r16(tp / r16(r16(tp + fp) + e))
    rec = r16(tp / r16(r16(tp + fn) + e))
    f1 = r16(r16(r16(prec * rec) * jnp.float32(2.0)) / r16(r16(prec + rec) + e))
    f1c = jnp.minimum(jnp.maximum(f1, e), jnp.float32(1.0))

    inv_n = jnp.float32(2.0 ** -22)
    rest = jnp.float32((_C * _L - _L) * 2.0 ** -23)
    mf1 = (jnp.sum(f1c) + rest) * inv_n
    out1 = jnp.sum(prec) * inv_n
    out2 = jnp.sum(rec) * inv_n

    lane = jax.lax.broadcasted_iota(jnp.int32, (8, 128), 1)
    mf1_16 = r16(jnp.where(lane == 0, mf1, 0.0))
    o_ref[...] = jnp.where(lane == 0, jnp.float32(1.0) - mf1_16,
                           jnp.where(lane == 1, out1, out2))


def kernel(y_pred, y_true):
    yp = y_pred.reshape(_B * _C, _L)
    vals, keys, ypcol = pl.pallas_call(
        _topk_body,
        grid=(_B * _C // _RB,),
        in_specs=[pl.BlockSpec((_RB, _L), lambda i: (i, 0))],
        out_specs=[pl.BlockSpec((_RB, 1), lambda i: (i, 0)),
                   pl.BlockSpec((_RB, 1), lambda i: (i, 0)),
                   pl.BlockSpec((_RB, 1), lambda i: (i, 0))],
        out_shape=[jax.ShapeDtypeStruct((_B * _C, 1), jnp.float32),
                   jax.ShapeDtypeStruct((_B * _C, 1), jnp.int32),
                   jax.ShapeDtypeStruct((_B * _C, 1), jnp.float32)],
    )(yp)

    sk, sv = jax.lax.sort_key_val(keys.reshape(_B * _C), vals.reshape(_B * _C),
                                  is_stable=False)

    skm = sk.reshape(_NCH, _CHUNK)
    svm = sv.reshape(_NCH, _CHUNK)
    yp0 = ypcol.reshape(_B, _C)[:, :_L].reshape(_B * 8, 128)
    yt0 = y_true[:, :128]

    o = pl.pallas_call(
        _stats_body,
        out_shape=jax.ShapeDtypeStruct((8, 128), jnp.float32),
    )(skm, svm, yp0, yt0)

    return (o[0, 0].astype(jnp.float16),
            o[0, 1].astype(jnp.float16),
            o[0, 2].astype(jnp.float16))
